# baseline (speedup 1.0000x reference)
import functools
import jax
import jax.numpy as jnp
from jax import lax
from jax.experimental import pallas as pl
from jax.experimental.pallas import tpu as pltpu
from jax.experimental.pallas import tpu_sc as plsc

_NW = 32
_NBINS = 256


def _lrelu(x):
    return jnp.where(x >= 0, x, 0.01 * x)


def _argmax_first(x):
    m = jnp.max(x, axis=1, keepdims=True)
    c = x.shape[1]
    iota = lax.broadcasted_iota(jnp.int32, x.shape, 1)
    return jnp.min(jnp.where(x == m, iota, c), axis=1,
                   keepdims=True).astype(jnp.int32)


def _dot(a, b):
    return lax.dot_general(a, b, (((1,), (0,)), ((), ())),
                           preferred_element_type=jnp.float32)


def _dot_t(a, b):
    return lax.dot_general(a, b, (((1,), (1,)), ((), ())),
                           preferred_element_type=jnp.float32)


def _front_body(xf_ref,
                wfcat, bfcat, bb2_w, bb2_b, bb3_w, bb3_b,
                msk2_w, msk2_b, msk3_w, msk3_b,
                c10_w, c10_b, c20_w, c20_b, c30_w, c30_b,
                w11cat, b11cat, w21cat, c21_b, w31cat, c31_b,
                mask_ref, xl_ref, xr_ref, inds_ref):
    xf = xf_ref[...]
    f0 = _lrelu(_dot(xf, wfcat[...]) + bfcat[...])
    x = f0[:, 0:128]
    m = f0[:, 128:160]
    xr_ref[...] = f0[:, 160:288]
    x = _lrelu(_dot(x, bb2_w[...]) + bb2_b[...])
    xl = _lrelu(_dot(x, bb3_w[...]) + bb3_b[...])
    xl_ref[...] = xl
    m = _lrelu(_dot(m, msk2_w[...]) + msk2_b[...])
    m = _lrelu(_dot(m, msk3_w[...]) + msk3_b[...])
    mask_ref[...] = m
    x = _lrelu(_dot(xl, c10_w[...]) + c10_b[...])
    x = _lrelu(_dot(x, c20_w[...]) + c20_b[...])
    x1 = _dot(x, c30_w[...]) + c30_b[...]
    inds1 = _argmax_first(x1)
    t = xf.shape[0]
    h1a = _lrelu(_dot(xl, w11cat[...]) + b11cat[...])
    b2sel = jnp.zeros((t, 32), jnp.float32)
    b3sel = jnp.zeros((t, 32), jnp.float32)
    for e in range(16):
        b2sel = jnp.where(inds1 == e, c21_b[e:e + 1, :], b2sel)
        b3sel = jnp.where(inds1 == e, c31_b[e:e + 1, :], b3sel)
    ecol = lax.broadcasted_iota(jnp.int32, (t, 128), 1) // 32
    h2 = jnp.zeros((t, 32), jnp.float32)
    for g in range(4):
        sel = (ecol + 4 * g) == inds1
        x4 = jnp.where(sel, h1a[:, g * 128:(g + 1) * 128], 0.0)
        h2 = h2 + _dot(x4, w21cat[g])
    h2 = _lrelu(h2 + b2sel)
    h2t = jnp.tile(h2, (1, 4))
    x2 = jnp.zeros((t, 32), jnp.float32)
    for g in range(4):
        sel = (ecol + 4 * g) == inds1
        x4 = jnp.where(sel, h2t, 0.0)
        x2 = x2 + _dot(x4, w31cat[g])
    x2 = x2 + b3sel
    inds2 = _argmax_first(x2)
    inds_ref[...] = jnp.clip(inds1 * 16 + (inds2 - 8), 0, 255)


def _run_front(xf, p, ta):
    n = xf.shape[0]
    nb = n // ta
    full = lambda a: pl.BlockSpec(a.shape, lambda g: (0,) * a.ndim)
    args = [p['wfcat'], p['bfcat'], p['bb2_w'], p['bb2_b'], p['bb3_w'],
            p['bb3_b'], p['msk2_w'], p['msk2_b'],
            p['msk3_w'], p['msk3_b'], p['c10_w'], p['c10_b'], p['c20_w'],
            p['c20_b'], p['c30_w'], p['c30_b'], p['w11cat'], p['b11cat'],
            p['w21cat'], p['c21_b'], p['w31cat'], p['c31_b']]
    out = pl.pallas_call(
        _front_body,
        grid=(nb,),
        in_specs=[pl.BlockSpec((ta, 128), lambda g: (g, 0))] +
                 [full(a) for a in args],
        out_specs=[
            pl.BlockSpec((ta, 1), lambda g: (g, 0)),
            pl.BlockSpec((ta, 128), lambda g: (g, 0)),
            pl.BlockSpec((ta, 128), lambda g: (g, 0)),
            pl.BlockSpec((ta, 1), lambda g: (g, 0)),
        ],
        out_shape=[
            jax.ShapeDtypeStruct((n, 1), jnp.float32),
            jax.ShapeDtypeStruct((n, 128), jnp.float32),
            jax.ShapeDtypeStruct((n, 128), jnp.float32),
            jax.ShapeDtypeStruct((n, 1), jnp.int32),
        ],
    )(xf, *args)
    return out


def _wid():
    return lax.axis_index("s") * 2 + lax.axis_index("c")


def _sc_mesh():
    return plsc.VectorSubcoreMesh(core_axis_name="c", subcore_axis_name="s")


def _count_base():
    cnt, _ = plsc.scan_count(lax.iota(jnp.int32, 16))
    return cnt


def _sc_hist(inds):
    n = inds.shape[0]
    ch = n // _NW
    ng = ch // 16

    @functools.partial(
        pl.kernel,
        mesh=_sc_mesh(),
        compiler_params=pltpu.CompilerParams(needs_layout_passes=False),
        out_type=jax.ShapeDtypeStruct((_NW, _NBINS), jnp.int32),
        scratch_types=[pltpu.VMEM((ch,), jnp.int32),
                       pltpu.VMEM((_NBINS,), jnp.int32)],
    )
    def k(inds_hbm, ph_hbm, idx_v, hist_v):
        w = _wid()
        pltpu.sync_copy(inds_hbm.at[pl.ds(w * ch, ch)], idx_v)
        zero = jnp.zeros((16,), jnp.int32)
        for v in range(_NBINS // 16):
            hist_v[pl.ds(v * 16, 16)] = zero
        cb = _count_base()

        def body(g, carry):
            idx = idx_v[pl.ds(g * 16, 16)]
            cnt, last = plsc.scan_count(idx)
            plsc.addupdate_scatter(hist_v, [idx], cnt - cb + 1, mask=last)
            return carry

        lax.fori_loop(0, ng, body, 0)
        pltpu.sync_copy(hist_v, ph_hbm.at[w])

    return k(inds)


def _sc_scatter(ph, inds, xl, xr, nsub, sub):
    n = inds.shape[0]
    ch = n // _NW
    ng = ch // 16
    gps = sub // 16

    @functools.partial(
        pl.kernel,
        mesh=_sc_mesh(),
        compiler_params=pltpu.CompilerParams(needs_layout_passes=False),
        out_type=[jax.ShapeDtypeStruct((_NW, nsub, sub), jnp.int32),
                  jax.ShapeDtypeStruct((n, 128), jnp.float32),
                  jax.ShapeDtypeStruct((n, 128), jnp.float32)],
        scratch_types=[pltpu.VMEM((_NW, _NBINS), jnp.int32),
                       pltpu.VMEM((_NBINS,), jnp.int32),
                       pltpu.VMEM((ch,), jnp.int32),
                       pltpu.VMEM((nsub, sub), jnp.int32),
                       pltpu.VMEM((sub, 128), jnp.float32),
                       pltpu.VMEM((sub, 128), jnp.float32),
                       pltpu.VMEM((sub, 128), jnp.float32),
                       pltpu.VMEM((sub, 128), jnp.float32),
                       pltpu.SemaphoreType.DMA,
                       pltpu.SemaphoreType.DMA],
    )
    def k(ph_hbm, inds_hbm, xl_hbm, xr_hbm,
          pos_hbm, xls_hbm, xrs_hbm,
          ph_v, next_v, idx_v, pos_v, rl0, rr0, rl1, rr1, sem_g, sem_s):
        w = _wid()
        base = w * ch
        pltpu.sync_copy(ph_hbm, ph_v)
        carry = jnp.zeros((16,), jnp.int32)
        for v in range(_NBINS // 16):
            tot = jnp.zeros((16,), jnp.int32)
            below = jnp.zeros((16,), jnp.int32)
            for s in range(_NW):
                row = ph_v[s, pl.ds(v * 16, 16)]
                tot = tot + row
                below = below + row * (s < w).astype(jnp.int32)
            ex = carry + plsc.cumsum(tot) - tot
            next_v[pl.ds(v * 16, 16)] = ex + below
            carry = carry + jnp.full((16,), jnp.sum(tot), jnp.int32)
        pltpu.sync_copy(inds_hbm.at[pl.ds(base, ch)], idx_v)
        cb = _count_base()

        def body(g, carry2):
            idx = idx_v[pl.ds(g * 16, 16)]
            cnt, last = plsc.scan_count(idx)
            nx = plsc.load_gather(next_v, [idx])
            pos = nx + (cnt - cb)
            plsc.store_scatter(next_v, [idx], pos + 1, mask=last)
            r = g // gps
            c = (g % gps) * 16
            pos_v[r, pl.ds(c, 16)] = pos
            return carry2

        lax.fori_loop(0, ng, body, 0)
        pltpu.sync_copy(pos_v, pos_hbm.at[w])
        bufs = [(rl0, rr0), (rl1, rr1)]
        sc_d = [None, None]
        for j in range(nsub):
            b = j % 2
            rl, rr = bufs[b]
            if sc_d[b] is not None:
                sc_d[b][0].wait()
                sc_d[b][1].wait()
            gl = pltpu.async_copy(xl_hbm.at[pl.ds(base + j * sub, sub)],
                                  rl, sem_g)
            gr = pltpu.async_copy(xr_hbm.at[pl.ds(base + j * sub, sub)],
                                  rr, sem_g)
            gl.wait()
            gr.wait()
            sc_d[b] = (pltpu.async_copy(rl, xls_hbm.at[pos_v.at[j]], sem_s),
                       pltpu.async_copy(rr, xrs_hbm.at[pos_v.at[j]], sem_s))
        for d in sc_d:
            if d is not None:
                d[0].wait()
                d[1].wait()

    return k(ph, inds, xl, xr)


def _sc_unsort(val, pos, nsub, sub):
    @functools.partial(
        pl.kernel,
        mesh=_sc_mesh(),
        compiler_params=pltpu.CompilerParams(needs_layout_passes=False),
        out_type=jax.ShapeDtypeStruct((_NW, nsub, sub), jnp.float32),
        scratch_types=[pltpu.VMEM((nsub, sub), jnp.int32),
                       pltpu.VMEM((nsub, sub), jnp.float32),
                       pltpu.SemaphoreType.DMA],
    )
    def k(val_hbm, pos_hbm, out_hbm, pos_v, out_v, sem):
        w = _wid()
        pltpu.sync_copy(pos_hbm.at[w], pos_v)
        ds = [pltpu.async_copy(val_hbm.at[pos_v.at[j]], out_v.at[j], sem)
              for j in range(nsub)]
        for d in ds:
            d.wait()
        pltpu.sync_copy(out_v, out_hbm.at[w])

    return k(val, pos)


def _prefix_body(ph_ref, cum_ref):
    tot = jnp.sum(ph_ref[...].astype(jnp.float32), axis=0, keepdims=True)
    ii = lax.broadcasted_iota(jnp.int32, (_NBINS, _NBINS), 0)
    jj = lax.broadcasted_iota(jnp.int32, (_NBINS, _NBINS), 1)
    cum_ref[...] = lax.dot_general(tot, (ii <= jj).astype(jnp.float32),
                                   (((1,), (0,)), ((), ())),
                                   preferred_element_type=jnp.float32,
                                   precision=lax.Precision.HIGHEST)


def _run_prefix(ph):
    return pl.pallas_call(
        _prefix_body,
        out_shape=jax.ShapeDtypeStruct((1, _NBINS), jnp.float32),
    )(ph)


def _stage3_body(cum_ref, xls_ref, xrs_ref,
                 W12, b12, W22, b22, W32, b32,
                 r2W, r2b, r3Wr, r3br,
                 val_ref):
    t = xls_ref.shape[0]
    g = pl.program_id(0)
    cum = cum_ref[...]
    pvec = (g * t + lax.broadcasted_iota(jnp.int32, (t, 1), 0)
            ).astype(jnp.float32)
    inds = jnp.sum((cum <= pvec).astype(jnp.int32), axis=1, keepdims=True)
    xl = xls_ref[...]
    e0 = jnp.min(inds)
    e1 = jnp.max(inds)

    def chain_body(e, ind_tok):
        h = _lrelu(_dot(xl, W12[e]) + b12[pl.ds(e, 1), :])
        h = _lrelu(_dot(h, W22[e]) + b22[pl.ds(e, 1), :])
        h = _dot(h, W32[e]) + b32[pl.ds(e, 1), :]
        inds3 = _argmax_first(h)
        it = jnp.clip(e * 16 + (inds3 - 8), 0, 4095)
        return jnp.where(inds == e, it, ind_tok)

    ind_tok = lax.fori_loop(e0, e1 + 1, chain_body,
                            jnp.zeros((t, 1), jnp.int32))

    xr = xrs_ref[...]
    sup = ind_tok // 512
    smin = jnp.min(sup)
    smax = jnp.max(sup)

    def r2_body(s, x32):
        h = _lrelu(_dot(xr, r2W[s]) + r2b[pl.ds(s, 1), :])
        return jnp.where(sup == s, h, x32)

    x32 = lax.fori_loop(smin, smax + 1, r2_body,
                        jnp.zeros((t, 32), jnp.float32))

    def r3_body(e, racc):
        start = jnp.clip(e * 16 - 8, 0, 4096 - 32)
        wsl = r3Wr[pl.ds(start, 32), :]
        bsl = r3br[pl.ds(start, 32), :]
        z = _dot_t(x32, wsl)
        local = ind_tok - start
        oh = lax.broadcasted_iota(jnp.int32, (t, 32), 1) == local
        rr = (jnp.sum(jnp.where(oh, z, 0.0), axis=1, keepdims=True)
              + _dot(oh.astype(jnp.float32), bsl))
        return jnp.where(inds == e, rr, racc)

    r = lax.fori_loop(e0, e1 + 1, r3_body, jnp.zeros((t, 1), jnp.float32))
    val_ref[...] = (ind_tok.astype(jnp.float32) + r) * (1.0 / 4096.0)


def _run_stage3(cum, xls, xrs, p, tc):
    n = xls.shape[0]
    nb = n // tc
    r3wr = p['r3_W'].reshape(4096, 32)
    r3br = p['r3_b'].reshape(4096, 1)
    full = lambda a: pl.BlockSpec(a.shape, lambda g: (0,) * a.ndim)
    args = [p['c12_W'], p['c12_b'], p['c22_W'], p['c22_b'], p['c32_W'],
            p['c32_b'], p['r2_W'], p['r2_b'], r3wr, r3br]
    val = pl.pallas_call(
        _stage3_body,
        grid=(nb,),
        in_specs=[full(cum)] +
                 [pl.BlockSpec((tc, 128), lambda g: (g, 0)),
                  pl.BlockSpec((tc, 128), lambda g: (g, 0))] +
                 [full(a) for a in args],
        out_specs=pl.BlockSpec((tc, 1), lambda g: (g, 0)),
        out_shape=jax.ShapeDtypeStruct((n, 1), jnp.float32),
    )(cum, xls, xrs, *args)
    return val.reshape(n)


def kernel(x_in, bb1_w, bb1_b, bb2_w, bb2_b, bb3_w, bb3_b,
           msk1_w, msk1_b, msk2_w, msk2_b, msk3_w, msk3_b,
           c10_w, c10_b, c20_w, c20_b, c30_w, c30_b,
           c11_W, c11_b, c21_W, c21_b, c31_W, c31_b,
           c12_W, c12_b, c22_W, c22_b, c32_W, c32_b,
           r1_w, r1_b, r2_W, r2_b, r3_W, r3_b):
    B, C, H, W = x_in.shape
    n = B * H * W
    ta = 3584 if n % 3584 == 0 else 128
    tc = 1024 if n % 1024 == 0 else 128
    ch = n // _NW
    sub = 112 if ch % 112 == 0 else 64
    nsub = ch // sub
    xf = jnp.transpose(x_in, (0, 2, 3, 1)).reshape(n, C)
    p = dict(
        bb1_w=bb1_w, bb1_b=bb1_b.reshape(1, -1),
        bb2_w=bb2_w, bb2_b=bb2_b.reshape(1, -1),
        bb3_w=bb3_w, bb3_b=bb3_b.reshape(1, -1),
        msk1_w=msk1_w, msk1_b=msk1_b.reshape(1, -1),
        msk2_w=msk2_w, msk2_b=msk2_b.reshape(1, -1),
        msk3_w=msk3_w, msk3_b=msk3_b.reshape(1, -1),
        c10_w=c10_w, c10_b=c10_b.reshape(1, -1),
        c20_w=c20_w, c20_b=c20_b.reshape(1, -1),
        c30_w=c30_w, c30_b=c30_b.reshape(1, -1),
        wfcat=jnp.concatenate([bb1_w, msk1_w, r1_w], axis=1),
        bfcat=jnp.concatenate([bb1_b, msk1_b, r1_b]).reshape(1, 288),
        w11cat=jnp.transpose(c11_W, (1, 0, 2)).reshape(128, 512),
        b11cat=c11_b.reshape(1, 512),
        w21cat=c21_W.reshape(4, 128, 32), c21_b=c21_b,
        w31cat=c31_W.reshape(4, 128, 32), c31_b=c31_b,
        c12_W=c12_W, c12_b=c12_b, c22_W=c22_W, c22_b=c22_b,
        c32_W=c32_W, c32_b=c32_b,
        r1_w=r1_w, r1_b=r1_b.reshape(1, -1),
        r2_W=r2_W, r2_b=r2_b, r3_W=r3_W, r3_b=r3_b,
    )
    mask2, xl, xr, inds12 = _run_front(xf, p, ta)
    inds_flat = inds12.reshape(n)

    ph = _sc_hist(inds_flat)
    pos, xls, xrs = _sc_scatter(ph, inds_flat, xl, xr, nsub, sub)

    cum = _run_prefix(ph)
    val_sorted = _run_stage3(cum, xls, xrs, p, tc)

    out_flat = _sc_unsort(val_sorted, pos, nsub, sub).reshape(n)
    out = out_flat.reshape(B, 1, H, W)
    mask = mask2.reshape(B, 1, H, W)
    return out, mask


# speedup vs baseline: 1.0058x; 1.0058x over previous
import functools
import jax
import jax.numpy as jnp
from jax import lax
from jax.experimental import pallas as pl
from jax.experimental.pallas import tpu as pltpu
from jax.experimental.pallas import tpu_sc as plsc

_NW = 32
_NBINS = 256


def _lrelu(x):
    return jnp.where(x >= 0, x, 0.01 * x)


def _argmax_first(x):
    m = jnp.max(x, axis=1, keepdims=True)
    c = x.shape[1]
    iota = lax.broadcasted_iota(jnp.int32, x.shape, 1)
    return jnp.min(jnp.where(x == m, iota, c), axis=1,
                   keepdims=True).astype(jnp.int32)


def _dot(a, b):
    return lax.dot_general(a, b, (((1,), (0,)), ((), ())),
                           preferred_element_type=jnp.float32)


def _dot_t(a, b):
    return lax.dot_general(a, b, (((1,), (1,)), ((), ())),
                           preferred_element_type=jnp.float32)


def _front_body(xf_ref,
                wfcat, bfcat, bb2_w, bb2_b, bb3_w, bb3_b,
                msk2_w, msk2_b, msk3_w, msk3_b,
                c10_w, c10_b, c20_w, c20_b, c30_w, c30_b,
                w11cat, b11cat, w21cat, c21_b, w31cat, c31_b,
                mask_ref, xl_ref, xr_ref, inds_ref):
    xf = xf_ref[...]
    f0 = _lrelu(_dot(xf, wfcat[...]) + bfcat[...])
    x = f0[:, 0:128]
    m = f0[:, 128:160]
    xr_ref[...] = f0[:, 160:288]
    x = _lrelu(_dot(x, bb2_w[...]) + bb2_b[...])
    xl = _lrelu(_dot(x, bb3_w[...]) + bb3_b[...])
    xl_ref[...] = xl
    m = _lrelu(_dot(m, msk2_w[...]) + msk2_b[...])
    m = _lrelu(_dot(m, msk3_w[...]) + msk3_b[...])
    mask_ref[...] = m
    x = _lrelu(_dot(xl, c10_w[...]) + c10_b[...])
    x = _lrelu(_dot(x, c20_w[...]) + c20_b[...])
    x1 = _dot(x, c30_w[...]) + c30_b[...]
    inds1 = _argmax_first(x1)
    t = xf.shape[0]
    h1a = _lrelu(_dot(xl, w11cat[...]) + b11cat[...])
    b2sel = jnp.zeros((t, 32), jnp.float32)
    b3sel = jnp.zeros((t, 32), jnp.float32)
    for e in range(16):
        b2sel = jnp.where(inds1 == e, c21_b[e:e + 1, :], b2sel)
        b3sel = jnp.where(inds1 == e, c31_b[e:e + 1, :], b3sel)
    ecol = lax.broadcasted_iota(jnp.int32, (t, 128), 1) // 32
    h2 = jnp.zeros((t, 32), jnp.float32)
    for g in range(4):
        sel = (ecol + 4 * g) == inds1
        x4 = jnp.where(sel, h1a[:, g * 128:(g + 1) * 128], 0.0)
        h2 = h2 + _dot(x4, w21cat[g])
    h2 = _lrelu(h2 + b2sel)
    h2t = jnp.tile(h2, (1, 4))
    x2 = jnp.zeros((t, 32), jnp.float32)
    for g in range(4):
        sel = (ecol + 4 * g) == inds1
        x4 = jnp.where(sel, h2t, 0.0)
        x2 = x2 + _dot(x4, w31cat[g])
    x2 = x2 + b3sel
    inds2 = _argmax_first(x2)
    inds_ref[...] = jnp.clip(inds1 * 16 + (inds2 - 8), 0, 255)


def _run_front(xf, p, ta):
    n = xf.shape[0]
    nb = n // ta
    full = lambda a: pl.BlockSpec(a.shape, lambda g: (0,) * a.ndim)
    args = [p['wfcat'], p['bfcat'], p['bb2_w'], p['bb2_b'], p['bb3_w'],
            p['bb3_b'], p['msk2_w'], p['msk2_b'],
            p['msk3_w'], p['msk3_b'], p['c10_w'], p['c10_b'], p['c20_w'],
            p['c20_b'], p['c30_w'], p['c30_b'], p['w11cat'], p['b11cat'],
            p['w21cat'], p['c21_b'], p['w31cat'], p['c31_b']]
    out = pl.pallas_call(
        _front_body,
        grid=(nb,),
        in_specs=[pl.BlockSpec((ta, 128), lambda g: (g, 0))] +
                 [full(a) for a in args],
        out_specs=[
            pl.BlockSpec((ta, 1), lambda g: (g, 0)),
            pl.BlockSpec((ta, 128), lambda g: (g, 0)),
            pl.BlockSpec((ta, 128), lambda g: (g, 0)),
            pl.BlockSpec((ta, 1), lambda g: (g, 0)),
        ],
        out_shape=[
            jax.ShapeDtypeStruct((n, 1), jnp.float32),
            jax.ShapeDtypeStruct((n, 128), jnp.float32),
            jax.ShapeDtypeStruct((n, 128), jnp.float32),
            jax.ShapeDtypeStruct((n, 1), jnp.int32),
        ],
    )(xf, *args)
    return out


def _wid():
    return lax.axis_index("s") * 2 + lax.axis_index("c")


def _sc_mesh():
    return plsc.VectorSubcoreMesh(core_axis_name="c", subcore_axis_name="s")


def _count_base():
    cnt, _ = plsc.scan_count(lax.iota(jnp.int32, 16))
    return cnt


def _sc_hist(inds):
    n = inds.shape[0]
    ch = n // _NW
    ng = ch // 16

    @functools.partial(
        pl.kernel,
        mesh=_sc_mesh(),
        compiler_params=pltpu.CompilerParams(needs_layout_passes=False),
        out_type=jax.ShapeDtypeStruct((_NW, _NBINS), jnp.int32),
        scratch_types=[pltpu.VMEM((ch,), jnp.int32),
                       pltpu.VMEM((_NBINS,), jnp.int32)],
    )
    def k(inds_hbm, ph_hbm, idx_v, hist_v):
        w = _wid()
        pltpu.sync_copy(inds_hbm.at[pl.ds(w * ch, ch)], idx_v)
        zero = jnp.zeros((16,), jnp.int32)
        for v in range(_NBINS // 16):
            hist_v[pl.ds(v * 16, 16)] = zero
        cb = _count_base()

        def body(g, carry):
            idx = idx_v[pl.ds(g * 16, 16)]
            cnt, last = plsc.scan_count(idx)
            plsc.addupdate_scatter(hist_v, [idx], cnt - cb + 1, mask=last)
            return carry

        lax.fori_loop(0, ng, body, 0)
        pltpu.sync_copy(hist_v, ph_hbm.at[w])

    return k(inds)


def _sc_scatter(ph, inds, xl, xr, nsub, sub):
    n = inds.shape[0]
    ch = n // _NW
    ng = ch // 16
    gps = sub // 16

    @functools.partial(
        pl.kernel,
        mesh=_sc_mesh(),
        compiler_params=pltpu.CompilerParams(needs_layout_passes=False),
        out_type=[jax.ShapeDtypeStruct((_NW, nsub, sub), jnp.int32),
                  jax.ShapeDtypeStruct((n, 128), jnp.float32),
                  jax.ShapeDtypeStruct((n, 128), jnp.float32)],
        scratch_types=[pltpu.VMEM((_NW, _NBINS), jnp.int32),
                       pltpu.VMEM((_NBINS,), jnp.int32),
                       pltpu.VMEM((ch,), jnp.int32),
                       pltpu.VMEM((nsub, sub), jnp.int32),
                       pltpu.VMEM((sub, 128), jnp.float32),
                       pltpu.VMEM((sub, 128), jnp.float32),
                       pltpu.VMEM((sub, 128), jnp.float32),
                       pltpu.VMEM((sub, 128), jnp.float32),
                       pltpu.SemaphoreType.DMA,
                       pltpu.SemaphoreType.DMA],
    )
    def k(ph_hbm, inds_hbm, xl_hbm, xr_hbm,
          pos_hbm, xls_hbm, xrs_hbm,
          ph_v, next_v, idx_v, pos_v, rl0, rr0, rl1, rr1, sem_g, sem_s):
        w = _wid()
        base = w * ch
        pltpu.sync_copy(ph_hbm, ph_v)
        carry = jnp.zeros((16,), jnp.int32)
        for v in range(_NBINS // 16):
            tot = jnp.zeros((16,), jnp.int32)
            below = jnp.zeros((16,), jnp.int32)
            for s in range(_NW):
                row = ph_v[s, pl.ds(v * 16, 16)]
                tot = tot + row
                below = below + row * (s < w).astype(jnp.int32)
            ex = carry + plsc.cumsum(tot) - tot
            next_v[pl.ds(v * 16, 16)] = ex + below
            carry = carry + jnp.full((16,), jnp.sum(tot), jnp.int32)
        pltpu.sync_copy(inds_hbm.at[pl.ds(base, ch)], idx_v)
        cb = _count_base()

        def body(g, carry2):
            idx = idx_v[pl.ds(g * 16, 16)]
            cnt, last = plsc.scan_count(idx)
            nx = plsc.load_gather(next_v, [idx])
            pos = nx + (cnt - cb)
            plsc.store_scatter(next_v, [idx], pos + 1, mask=last)
            r = g // gps
            c = (g % gps) * 16
            pos_v[r, pl.ds(c, 16)] = pos
            return carry2

        lax.fori_loop(0, ng, body, 0)
        pltpu.sync_copy(pos_v, pos_hbm.at[w])
        bufs = [(rl0, rr0), (rl1, rr1)]
        sc_d = [None, None]
        for j in range(nsub):
            b = j % 2
            rl, rr = bufs[b]
            if sc_d[b] is not None:
                sc_d[b][0].wait()
                sc_d[b][1].wait()
            gl = pltpu.async_copy(xl_hbm.at[pl.ds(base + j * sub, sub)],
                                  rl, sem_g)
            gr = pltpu.async_copy(xr_hbm.at[pl.ds(base + j * sub, sub)],
                                  rr, sem_g)
            gl.wait()
            gr.wait()
            sc_d[b] = (pltpu.async_copy(rl, xls_hbm.at[pos_v.at[j]], sem_s),
                       pltpu.async_copy(rr, xrs_hbm.at[pos_v.at[j]], sem_s))
        for d in sc_d:
            if d is not None:
                d[0].wait()
                d[1].wait()

    return k(ph, inds, xl, xr)


def _sc_unsort(val, pos, nsub, sub):
    @functools.partial(
        pl.kernel,
        mesh=_sc_mesh(),
        compiler_params=pltpu.CompilerParams(needs_layout_passes=False),
        out_type=jax.ShapeDtypeStruct((_NW, nsub, sub), jnp.float32),
        scratch_types=[pltpu.VMEM((nsub, sub), jnp.int32),
                       pltpu.VMEM((nsub, sub), jnp.float32),
                       pltpu.SemaphoreType.DMA],
    )
    def k(val_hbm, pos_hbm, out_hbm, pos_v, out_v, sem):
        w = _wid()
        pltpu.sync_copy(pos_hbm.at[w], pos_v)
        ds = [pltpu.async_copy(val_hbm.at[pos_v.at[j]], out_v.at[j], sem)
              for j in range(nsub)]
        for d in ds:
            d.wait()
        pltpu.sync_copy(out_v, out_hbm.at[w])

    return k(val, pos)


def _prefix_body(ph_ref, cum_ref):
    tot = jnp.sum(ph_ref[...].astype(jnp.float32), axis=0, keepdims=True)
    ii = lax.broadcasted_iota(jnp.int32, (_NBINS, _NBINS), 0)
    jj = lax.broadcasted_iota(jnp.int32, (_NBINS, _NBINS), 1)
    cum_ref[...] = lax.dot_general(tot, (ii <= jj).astype(jnp.float32),
                                   (((1,), (0,)), ((), ())),
                                   preferred_element_type=jnp.float32,
                                   precision=lax.Precision.HIGHEST)


def _run_prefix(ph):
    return pl.pallas_call(
        _prefix_body,
        out_shape=jax.ShapeDtypeStruct((1, _NBINS), jnp.float32),
    )(ph)


def _stage3_body(cum_ref, xls_ref, xrs_ref,
                 W12, b12, W22, b22, W32, b32,
                 r2W, r2b, r3Wr, r3br,
                 val_ref):
    t = xls_ref.shape[0]
    g = pl.program_id(0)
    cum = cum_ref[...]
    pvec = (g * t + lax.broadcasted_iota(jnp.int32, (t, 1), 0)
            ).astype(jnp.float32)
    inds = jnp.sum((cum <= pvec).astype(jnp.int32), axis=1, keepdims=True)
    xl = xls_ref[...]
    e0 = jnp.min(inds)
    e1 = jnp.max(inds)

    def chain_body(e, ind_tok):
        h = _lrelu(_dot(xl, W12[e]) + b12[pl.ds(e, 1), :])
        h = _lrelu(_dot(h, W22[e]) + b22[pl.ds(e, 1), :])
        h = _dot(h, W32[e]) + b32[pl.ds(e, 1), :]
        inds3 = _argmax_first(h)
        it = jnp.clip(e * 16 + (inds3 - 8), 0, 4095)
        return jnp.where(inds == e, it, ind_tok)

    ind_tok = lax.fori_loop(e0, e1 + 1, chain_body,
                            jnp.zeros((t, 1), jnp.int32))

    xr = xrs_ref[...]
    sup = ind_tok // 512
    smin = jnp.min(sup)
    smax = jnp.max(sup)

    def r2_body(s, x32):
        h = _lrelu(_dot(xr, r2W[s]) + r2b[pl.ds(s, 1), :])
        return jnp.where(sup == s, h, x32)

    x32 = lax.fori_loop(smin, smax + 1, r2_body,
                        jnp.zeros((t, 32), jnp.float32))

    def r3_body(e, racc):
        start = jnp.clip(e * 16 - 8, 0, 4096 - 32)
        wsl = r3Wr[pl.ds(start, 32), :]
        bsl = r3br[pl.ds(start, 32), :]
        z = _dot_t(x32, wsl)
        local = ind_tok - start
        oh = lax.broadcasted_iota(jnp.int32, (t, 32), 1) == local
        rr = (jnp.sum(jnp.where(oh, z, 0.0), axis=1, keepdims=True)
              + _dot(oh.astype(jnp.float32), bsl))
        return jnp.where(inds == e, rr, racc)

    r = lax.fori_loop(e0, e1 + 1, r3_body, jnp.zeros((t, 1), jnp.float32))
    val_ref[...] = (ind_tok.astype(jnp.float32) + r) * (1.0 / 4096.0)


def _run_stage3(cum, xls, xrs, p, tc):
    n = xls.shape[0]
    nb = n // tc
    r3wr = p['r3_W'].reshape(4096, 32)
    r3br = p['r3_b'].reshape(4096, 1)
    full = lambda a: pl.BlockSpec(a.shape, lambda g: (0,) * a.ndim)
    args = [p['c12_W'], p['c12_b'], p['c22_W'], p['c22_b'], p['c32_W'],
            p['c32_b'], p['r2_W'], p['r2_b'], r3wr, r3br]
    val = pl.pallas_call(
        _stage3_body,
        grid=(nb,),
        in_specs=[full(cum)] +
                 [pl.BlockSpec((tc, 128), lambda g: (g, 0)),
                  pl.BlockSpec((tc, 128), lambda g: (g, 0))] +
                 [full(a) for a in args],
        out_specs=pl.BlockSpec((tc, 1), lambda g: (g, 0)),
        out_shape=jax.ShapeDtypeStruct((n, 1), jnp.float32),
    )(cum, xls, xrs, *args)
    return val.reshape(n)


def kernel(x_in, bb1_w, bb1_b, bb2_w, bb2_b, bb3_w, bb3_b,
           msk1_w, msk1_b, msk2_w, msk2_b, msk3_w, msk3_b,
           c10_w, c10_b, c20_w, c20_b, c30_w, c30_b,
           c11_W, c11_b, c21_W, c21_b, c31_W, c31_b,
           c12_W, c12_b, c22_W, c22_b, c32_W, c32_b,
           r1_w, r1_b, r2_W, r2_b, r3_W, r3_b):
    B, C, H, W = x_in.shape
    n = B * H * W
    ta = 2048 if n % 2048 == 0 else 128
    tc = 1024 if n % 1024 == 0 else 128
    ch = n // _NW
    sub = 112 if ch % 112 == 0 else 64
    nsub = ch // sub
    xf = jnp.transpose(x_in, (0, 2, 3, 1)).reshape(n, C)
    p = dict(
        bb1_w=bb1_w, bb1_b=bb1_b.reshape(1, -1),
        bb2_w=bb2_w, bb2_b=bb2_b.reshape(1, -1),
        bb3_w=bb3_w, bb3_b=bb3_b.reshape(1, -1),
        msk1_w=msk1_w, msk1_b=msk1_b.reshape(1, -1),
        msk2_w=msk2_w, msk2_b=msk2_b.reshape(1, -1),
        msk3_w=msk3_w, msk3_b=msk3_b.reshape(1, -1),
        c10_w=c10_w, c10_b=c10_b.reshape(1, -1),
        c20_w=c20_w, c20_b=c20_b.reshape(1, -1),
        c30_w=c30_w, c30_b=c30_b.reshape(1, -1),
        wfcat=jnp.concatenate([bb1_w, msk1_w, r1_w], axis=1),
        bfcat=jnp.concatenate([bb1_b, msk1_b, r1_b]).reshape(1, 288),
        w11cat=jnp.transpose(c11_W, (1, 0, 2)).reshape(128, 512),
        b11cat=c11_b.reshape(1, 512),
        w21cat=c21_W.reshape(4, 128, 32), c21_b=c21_b,
        w31cat=c31_W.reshape(4, 128, 32), c31_b=c31_b,
        c12_W=c12_W, c12_b=c12_b, c22_W=c22_W, c22_b=c22_b,
        c32_W=c32_W, c32_b=c32_b,
        r1_w=r1_w, r1_b=r1_b.reshape(1, -1),
        r2_W=r2_W, r2_b=r2_b, r3_W=r3_W, r3_b=r3_b,
    )
    mask2, xl, xr, inds12 = _run_front(xf, p, ta)
    inds_flat = inds12.reshape(n)

    ph = _sc_hist(inds_flat)
    pos, xls, xrs = _sc_scatter(ph, inds_flat, xl, xr, nsub, sub)

    cum = _run_prefix(ph)
    val_sorted = _run_stage3(cum, xls, xrs, p, tc)

    out_flat = _sc_unsort(val_sorted, pos, nsub, sub).reshape(n)
    out = out_flat.reshape(B, 1, H, W)
    mask = mask2.reshape(B, 1, H, W)
    return out, mask
